# baseline (speedup 1.0000x reference)
import functools

import jax
import jax.numpy as jnp
from jax import lax
from jax.experimental import pallas as pl
from jax.experimental.pallas import tpu as pltpu
from jax.experimental.pallas import tpu_sc as plsc

NP_ = 8000
NPP_ = 8192
NT_ = 2000
E_ = 160000
EL_ = 20000
H_ = 256
DIN_ = 1024

NC_ = 2
NS_ = 16

NSLAB_ = 10
SLAB_ROWS_ = 100
SLAB_W_ = SLAB_ROWS_ * NPP_
DUMP_W_ = 10240
TILE_W_ = SLAB_W_ // NS_
EPT_ = E_ // NS_
ZW_ = 6400
NZC_ = TILE_W_ // ZW_


def _adj_body(ed_hbm, es_hbm, a_out, a_sh, dvm, svm, idxv, onesv, zbuf, sem,
              osem):
    c = lax.axis_index("c")
    s = lax.axis_index("s")
    lane = lax.iota(jnp.int32, 16)

    pltpu.sync_copy(ed_hbm.at[pl.ds(s * EPT_, EPT_)], dvm)
    pltpu.sync_copy(es_hbm.at[pl.ds(s * EPT_, EPT_)], svm)

    def _fill_z(i, _):
        zbuf[pl.ds(i * 16, 16)] = jnp.zeros((16,), jnp.float32)
        return 0
    lax.fori_loop(0, ZW_ // 16, _fill_z, 0)

    def _fill_1(i, _):
        onesv[pl.ds(i * 16, 16)] = jnp.ones((16,), jnp.float32)
        return 0
    lax.fori_loop(0, EPT_ // 16, _fill_1, 0)

    for slab in range(NSLAB_):
        base_cell = (c * NSLAB_ + slab) * SLAB_W_

        def _step(i, _):
            d = dvm[pl.ds(i * 16, 16)]
            sv = svm[pl.ds(i * 16, 16)]
            rel = d * NPP_ + sv - base_cell
            m = (rel >= 0) & (rel < SLAB_W_)
            dummy = SLAB_W_ + i * 16 + lane
            idxv[pl.ds(i * 16, 16)] = jnp.where(m, rel, dummy)
            return 0
        lax.fori_loop(0, EPT_ // 16, _step, 0)

        if slab > 0:
            pltpu.make_async_copy(
                a_sh.at[pl.ds(s * TILE_W_, TILE_W_)],
                a_out.at[pl.ds(s * TILE_W_, TILE_W_)], osem).wait()
        zcs = [pltpu.async_copy(
                   zbuf, a_sh.at[pl.ds(s * TILE_W_ + k * ZW_, ZW_)], sem)
               for k in range(NZC_)]
        for h in zcs:
            h.wait()
        plsc.subcore_barrier()
        pltpu.sync_copy(onesv, a_sh.at[idxv], add=True)
        plsc.subcore_barrier()
        pltpu.async_copy(a_sh.at[pl.ds(s * TILE_W_, TILE_W_)],
                         a_out.at[pl.ds(base_cell + s * TILE_W_, TILE_W_)],
                         osem)
    pltpu.make_async_copy(
        a_sh.at[pl.ds(s * TILE_W_, TILE_W_)],
        a_out.at[pl.ds(s * TILE_W_, TILE_W_)], osem).wait()


def _build_adj(edge_dst, edge_src):
    kern = pl.kernel(
        _adj_body,
        out_type=jax.ShapeDtypeStruct((NT_ * NPP_,), jnp.float32),
        mesh=plsc.VectorSubcoreMesh(core_axis_name="c", subcore_axis_name="s"),
        scratch_types=[
            pltpu.VMEM_SHARED((SLAB_W_ + DUMP_W_,), jnp.float32),
            pltpu.VMEM((EPT_,), jnp.int32),
            pltpu.VMEM((EPT_,), jnp.int32),
            pltpu.VMEM((EPT_,), jnp.int32),
            pltpu.VMEM((EPT_,), jnp.float32),
            pltpu.VMEM((ZW_,), jnp.float32),
            pltpu.SemaphoreType.DMA,
            pltpu.SemaphoreType.DMA,
        ],
    )
    return kern(edge_dst, edge_src)


EL_P_ = 20480
CROWS_ = 32
LPT_ = EL_P_ // 32
NCH_ = LPT_ // CROWS_


def _lane_shuffle(x, idx):
    dnums = lax.GatherDimensionNumbers(
        offset_dims=(), collapsed_slice_dims=(0,), start_index_map=(0,))
    return lax.gather(x, idx[:, None], dnums, (1,),
                      mode=lax.GatherScatterMode.PROMISE_IN_BOUNDS)


def _cls_body(cat_hbm, ls_hbm, ld_hbm, o_hbm,
              idx0, idx1, rows0, rows1, rows2, rows3, obuf,
              gs0, gs1, gs2, gs3):
    c = lax.axis_index("c")
    s = lax.axis_index("s")
    wid = s * NC_ + c
    base = wid * LPT_
    lane = lax.iota(jnp.int32, 16)

    pltpu.sync_copy(ls_hbm.at[pl.ds(base, LPT_)], idx0.at[pl.ds(0, LPT_)])
    pltpu.sync_copy(ld_hbm.at[pl.ds(base, LPT_)], idx0.at[pl.ds(LPT_, LPT_)])

    def _mkidx(i, _):
        ch = i // (CROWS_ // 16)
        r = i % (CROWS_ // 16)
        p = idx0[pl.ds(ch * CROWS_ + r * 16, 16)]
        t = idx0[pl.ds(LPT_ + ch * CROWS_ + r * 16, 16)] + NPP_
        idx1[pl.ds(ch * 2 * CROWS_ + r * 16, 16)] = p
        idx1[pl.ds(ch * 2 * CROWS_ + CROWS_ + r * 16, 16)] = t
        return 0
    lax.fori_loop(0, NCH_ * (CROWS_ // 16), _mkidx, 0)

    def _fire(ch, rows, gs):
        pltpu.async_copy(
            cat_hbm.at[idx1.at[pl.ds(ch * 2 * CROWS_, 2 * CROWS_)]], rows, gs)

    def _drain(rows, gs):
        pltpu.make_async_copy(cat_hbm.at[pl.ds(0, 2 * CROWS_)], rows,
                              gs).wait()

    def _compute(ch, rows):
        for g in range(CROWS_ // 16):
            out_vec = jnp.zeros((16,), jnp.float32)
            for r16 in range(16):
                r = g * 16 + r16
                acc = rows[r, pl.ds(0, 16)] * rows[CROWS_ + r, pl.ds(0, 16)]
                for v in range(1, 16):
                    acc = acc + (rows[r, pl.ds(v * 16, 16)]
                                 * rows[CROWS_ + r, pl.ds(v * 16, 16)])
                for sh in (8, 4, 2, 1):
                    acc = acc + _lane_shuffle(acc, lane ^ sh)
                out_vec = jnp.where(lane == r16, acc, out_vec)
            out_vec = 1.0 / (1.0 + jnp.exp(-out_vec))
            obuf[pl.ds(ch * CROWS_ + g * 16, 16)] = out_vec

    bufs = ((rows0, gs0), (rows1, gs1), (rows2, gs2), (rows3, gs3))
    for b, (rb, gb) in enumerate(bufs):
        _fire(b, rb, gb)

    def _quad(i, _):
        i4 = 4 * i
        for b, (rb, gb) in enumerate(bufs):
            _drain(rb, gb)
            _compute(i4 + b, rb)
            @pl.when(i4 + b + 4 < NCH_)
            def _():
                _fire(i4 + b + 4, rb, gb)
        return 0
    lax.fori_loop(0, NCH_ // 4, _quad, 0)

    pltpu.sync_copy(obuf, o_hbm.at[pl.ds(base, LPT_)])


def _classifier(cat, label_src, label_dst):
    kern = pl.kernel(
        _cls_body,
        out_type=jax.ShapeDtypeStruct((EL_P_,), jnp.float32),
        mesh=plsc.VectorSubcoreMesh(core_axis_name="c", subcore_axis_name="s"),
        scratch_types=[
            pltpu.VMEM((2 * LPT_,), jnp.int32),
            pltpu.VMEM((2 * LPT_,), jnp.int32),
            pltpu.VMEM((2 * CROWS_, H_), jnp.float32),
            pltpu.VMEM((2 * CROWS_, H_), jnp.float32),
            pltpu.VMEM((2 * CROWS_, H_), jnp.float32),
            pltpu.VMEM((2 * CROWS_, H_), jnp.float32),
            pltpu.VMEM((LPT_,), jnp.float32),
            pltpu.SemaphoreType.DMA,
            pltpu.SemaphoreType.DMA,
            pltpu.SemaphoreType.DMA,
            pltpu.SemaphoreType.DMA,
        ],
    )
    ls = jnp.pad(label_src, (0, EL_P_ - EL_))
    ld = jnp.pad(label_dst, (0, EL_P_ - EL_))
    return kern(cat, ls, ld)[:EL_]


def _enc_body(x_ref, w_ref, b_ref, e_ref, o_ref):
    o_ref[...] = (jnp.dot(x_ref[...], w_ref[...],
                          preferred_element_type=jnp.float32)
                  + b_ref[...] + e_ref[...])


def _encoder(prot_x, lin_W, lin_b, prot_emb):
    bm = 1000
    grid = NP_ // bm
    return pl.pallas_call(
        _enc_body,
        grid=(grid,),
        in_specs=[
            pl.BlockSpec((bm, DIN_), lambda i: (i, 0)),
            pl.BlockSpec((DIN_, H_), lambda i: (0, 0)),
            pl.BlockSpec((H_,), lambda i: (0,)),
            pl.BlockSpec((bm, H_), lambda i: (i, 0)),
        ],
        out_specs=pl.BlockSpec((bm, H_), lambda i: (i, 0)),
        out_shape=jax.ShapeDtypeStruct((NP_, H_), jnp.float32),
    )(prot_x, lin_W, lin_b, prot_emb)


def _gnn_body(a_ref, xp_ref, te_ref,
              wlt_ref, blt_ref, wrt_ref, wlp_ref, blp_ref, wrp_ref,
              xt2_out, xp2_out, acc_ref, cnt_ref, xps_ref, xtc_ref, xcur_ref):
    l = pl.program_id(0)
    j = pl.program_id(1)
    nj = pl.num_programs(1)
    a = a_ref[...]

    @pl.when((l == 0) & (j == 0))
    def _():
        cnt_ref[...] = jnp.zeros_like(cnt_ref)
        xtc_ref[...] = te_ref[...]

    @pl.when(j == 0)
    def _():
        acc_ref[...] = jnp.zeros_like(acc_ref)

    @pl.when(l == 0)
    def _():
        xcur_ref[...] = xp_ref[...]
        cnt_ref[...] += jnp.sum(a, axis=1)

    @pl.when(l == 1)
    def _():
        xcur_ref[...] = xps_ref[pl.ds(j * BJ_, BJ_), :]

    xpj = xcur_ref[...]
    acc_ref[...] += jnp.dot(a, xpj, preferred_element_type=jnp.float32)

    mp = lax.dot_general(a, xtc_ref[...],
                         dimension_numbers=(((0,), (0,)), ((), ())),
                         preferred_element_type=jnp.float32)
    cntp = jnp.sum(a, axis=0)
    aggp = mp * (1.0 / jnp.maximum(cntp, 1.0))[:, None]
    wlp = wlp_ref[...][0]
    wrp = wrp_ref[...][0]
    rp = (jnp.dot(aggp, wlp, preferred_element_type=jnp.float32)
          + jnp.where(l == 0, blp_ref[...][0], blp_ref[...][1])
          + jnp.dot(xpj, wrp, preferred_element_type=jnp.float32))

    @pl.when(l == 0)
    def _():
        xps_ref[pl.ds(j * BJ_, BJ_), :] = jnp.maximum(rp, 0.0)

    @pl.when(l == 1)
    def _():
        xp2_out[...] = rp

    @pl.when(j == nj - 1)
    def _():
        inv = 1.0 / jnp.maximum(cnt_ref[...], 1.0)
        aggt = acc_ref[...] * inv[:, None]
        rt = (jnp.dot(aggt, wlt_ref[...][0], preferred_element_type=jnp.float32)
              + jnp.where(l == 0, blt_ref[...][0], blt_ref[...][1])
              + jnp.dot(xtc_ref[...], wrt_ref[...][0],
                        preferred_element_type=jnp.float32))

        @pl.when(l == 0)
        def _():
            xtc_ref[...] = jnp.maximum(rt, 0.0)

        @pl.when(l == 1)
        def _():
            xt2_out[...] = rt


BJ_ = 1024


def _gnn(A, XP, TE, WLT, BLT, WRT, WLP, BLP, WRP):
    grid = (2, NPP_ // BJ_)
    return pl.pallas_call(
        _gnn_body,
        grid=grid,
        in_specs=[
            pl.BlockSpec((NT_, BJ_), lambda l, j: (0, j)),
            pl.BlockSpec((BJ_, H_), lambda l, j: (j * (1 - l), 0)),
            pl.BlockSpec((NT_, H_), lambda l, j: (0, 0)),
            pl.BlockSpec((1, H_, H_), lambda l, j: (l, 0, 0)),
            pl.BlockSpec((2, H_), lambda l, j: (0, 0)),
            pl.BlockSpec((1, H_, H_), lambda l, j: (l, 0, 0)),
            pl.BlockSpec((1, H_, H_), lambda l, j: (l, 0, 0)),
            pl.BlockSpec((2, H_), lambda l, j: (0, 0)),
            pl.BlockSpec((1, H_, H_), lambda l, j: (l, 0, 0)),
        ],
        out_specs=[
            pl.BlockSpec((NT_, H_), lambda l, j: (0, 0)),
            pl.BlockSpec((BJ_, H_), lambda l, j: (j, 0)),
        ],
        out_shape=[
            jax.ShapeDtypeStruct((NT_, H_), jnp.float32),
            jax.ShapeDtypeStruct((NPP_, H_), jnp.float32),
        ],
        scratch_shapes=[
            pltpu.VMEM((NT_, H_), jnp.float32),
            pltpu.VMEM((NT_,), jnp.float32),
            pltpu.VMEM((NPP_, H_), jnp.float32),
            pltpu.VMEM((NT_, H_), jnp.float32),
            pltpu.VMEM((BJ_, H_), jnp.float32),
        ],
    )(A, XP, TE, WLT, BLT, WRT, WLP, BLP, WRP)


def kernel(prot_x, prot_node_id, term_node_id, edge_src, edge_dst,
           label_src, label_dst, lin_W, lin_b, prot_emb, term_emb,
           c1_pt_Wl, c1_pt_bl, c1_pt_Wr, c1_tp_Wl, c1_tp_bl, c1_tp_Wr,
           c2_pt_Wl, c2_pt_bl, c2_pt_Wr, c2_tp_Wl, c2_tp_bl, c2_tp_Wr):
    edge_src = edge_src.astype(jnp.int32)
    edge_dst = edge_dst.astype(jnp.int32)
    label_src = label_src.astype(jnp.int32)
    label_dst = label_dst.astype(jnp.int32)

    a_flat = _build_adj(edge_dst, edge_src)
    A = a_flat.reshape(NT_, NPP_)

    xp = jnp.pad(_encoder(prot_x, lin_W, lin_b, prot_emb),
                 ((0, NPP_ - NP_), (0, 0)))
    WLT = jnp.stack([c1_pt_Wl, c2_pt_Wl])
    BLT = jnp.stack([c1_pt_bl, c2_pt_bl])
    WRT = jnp.stack([c1_pt_Wr, c2_pt_Wr])
    WLP = jnp.stack([c1_tp_Wl, c2_tp_Wl])
    BLP = jnp.stack([c1_tp_bl, c2_tp_bl])
    WRP = jnp.stack([c1_tp_Wr, c2_tp_Wr])
    xt2, xp2 = _gnn(A, xp, term_emb, WLT, BLT, WRT, WLP, BLP, WRP)

    cat = jnp.concatenate([xp2, xt2], axis=0)
    return _classifier(cat, label_src, label_dst)


# speedup vs baseline: 1.0149x; 1.0149x over previous
import functools

import jax
import jax.numpy as jnp
from jax import lax
from jax.experimental import pallas as pl
from jax.experimental.pallas import tpu as pltpu
from jax.experimental.pallas import tpu_sc as plsc

NP_ = 8000
NPP_ = 8192
NT_ = 2000
E_ = 160000
EL_ = 20000
H_ = 256
DIN_ = 1024

NC_ = 2
NS_ = 16

NSLAB_ = 8
SLAB_ROWS_ = 125
SLAB_W_ = SLAB_ROWS_ * NPP_
DUMP_W_ = 10240
TILE_W_ = SLAB_W_ // NS_
EPT_ = E_ // NS_
ZW_ = 6400
NZC_ = TILE_W_ // ZW_


def _adj_body(ed_hbm, es_hbm, a_out, a_sh, dvm, svm, idxv, onesv, zbuf, sem,
              osem):
    c = lax.axis_index("c")
    s = lax.axis_index("s")
    lane = lax.iota(jnp.int32, 16)

    pltpu.sync_copy(ed_hbm.at[pl.ds(s * EPT_, EPT_)], dvm)
    pltpu.sync_copy(es_hbm.at[pl.ds(s * EPT_, EPT_)], svm)

    def _fill_z(i, _):
        zbuf[pl.ds(i * 16, 16)] = jnp.zeros((16,), jnp.float32)
        return 0
    lax.fori_loop(0, ZW_ // 16, _fill_z, 0)

    def _fill_1(i, _):
        onesv[pl.ds(i * 16, 16)] = jnp.ones((16,), jnp.float32)
        return 0
    lax.fori_loop(0, EPT_ // 16, _fill_1, 0)

    for slab in range(NSLAB_):
        base_cell = (c * NSLAB_ + slab) * SLAB_W_

        def _step(i, _):
            d = dvm[pl.ds(i * 16, 16)]
            sv = svm[pl.ds(i * 16, 16)]
            rel = d * NPP_ + sv - base_cell
            m = (rel >= 0) & (rel < SLAB_W_)
            dummy = SLAB_W_ + i * 16 + lane
            idxv[pl.ds(i * 16, 16)] = jnp.where(m, rel, dummy)
            return 0
        lax.fori_loop(0, EPT_ // 16, _step, 0)

        if slab > 0:
            pltpu.make_async_copy(
                a_sh.at[pl.ds(s * TILE_W_, TILE_W_)],
                a_out.at[pl.ds(s * TILE_W_, TILE_W_)], osem).wait()
        zcs = [pltpu.async_copy(
                   zbuf, a_sh.at[pl.ds(s * TILE_W_ + k * ZW_, ZW_)], sem)
               for k in range(NZC_)]
        for h in zcs:
            h.wait()
        plsc.subcore_barrier()
        pltpu.sync_copy(onesv, a_sh.at[idxv], add=True)
        plsc.subcore_barrier()
        pltpu.async_copy(a_sh.at[pl.ds(s * TILE_W_, TILE_W_)],
                         a_out.at[pl.ds(base_cell + s * TILE_W_, TILE_W_)],
                         osem)
    pltpu.make_async_copy(
        a_sh.at[pl.ds(s * TILE_W_, TILE_W_)],
        a_out.at[pl.ds(s * TILE_W_, TILE_W_)], osem).wait()


def _build_adj(edge_dst, edge_src):
    kern = pl.kernel(
        _adj_body,
        out_type=jax.ShapeDtypeStruct((NT_ * NPP_,), jnp.float32),
        mesh=plsc.VectorSubcoreMesh(core_axis_name="c", subcore_axis_name="s"),
        scratch_types=[
            pltpu.VMEM_SHARED((SLAB_W_ + DUMP_W_,), jnp.float32),
            pltpu.VMEM((EPT_,), jnp.int32),
            pltpu.VMEM((EPT_,), jnp.int32),
            pltpu.VMEM((EPT_,), jnp.int32),
            pltpu.VMEM((EPT_,), jnp.float32),
            pltpu.VMEM((ZW_,), jnp.float32),
            pltpu.SemaphoreType.DMA,
            pltpu.SemaphoreType.DMA,
        ],
    )
    return kern(edge_dst, edge_src)


EL_P_ = 20480
CROWS_ = 32
LPT_ = EL_P_ // 32
NCH_ = LPT_ // CROWS_


def _lane_shuffle(x, idx):
    dnums = lax.GatherDimensionNumbers(
        offset_dims=(), collapsed_slice_dims=(0,), start_index_map=(0,))
    return lax.gather(x, idx[:, None], dnums, (1,),
                      mode=lax.GatherScatterMode.PROMISE_IN_BOUNDS)


def _cls_body(cat_hbm, ls_hbm, ld_hbm, o_hbm,
              idx0, idx1, rows0, rows1, rows2, rows3, obuf,
              gs0, gs1, gs2, gs3):
    c = lax.axis_index("c")
    s = lax.axis_index("s")
    wid = s * NC_ + c
    base = wid * LPT_
    lane = lax.iota(jnp.int32, 16)

    pltpu.sync_copy(ls_hbm.at[pl.ds(base, LPT_)], idx0.at[pl.ds(0, LPT_)])
    pltpu.sync_copy(ld_hbm.at[pl.ds(base, LPT_)], idx0.at[pl.ds(LPT_, LPT_)])

    def _mkidx(i, _):
        ch = i // (CROWS_ // 16)
        r = i % (CROWS_ // 16)
        p = idx0[pl.ds(ch * CROWS_ + r * 16, 16)]
        t = idx0[pl.ds(LPT_ + ch * CROWS_ + r * 16, 16)] + NPP_
        idx1[pl.ds(ch * 2 * CROWS_ + r * 16, 16)] = p
        idx1[pl.ds(ch * 2 * CROWS_ + CROWS_ + r * 16, 16)] = t
        return 0
    lax.fori_loop(0, NCH_ * (CROWS_ // 16), _mkidx, 0)

    def _fire(ch, rows, gs):
        pltpu.async_copy(
            cat_hbm.at[idx1.at[pl.ds(ch * 2 * CROWS_, 2 * CROWS_)]], rows, gs)

    def _drain(rows, gs):
        pltpu.make_async_copy(cat_hbm.at[pl.ds(0, 2 * CROWS_)], rows,
                              gs).wait()

    def _compute(ch, rows):
        for g in range(CROWS_ // 16):
            out_vec = jnp.zeros((16,), jnp.float32)
            for r16 in range(16):
                r = g * 16 + r16
                acc = rows[r, pl.ds(0, 16)] * rows[CROWS_ + r, pl.ds(0, 16)]
                for v in range(1, 16):
                    acc = acc + (rows[r, pl.ds(v * 16, 16)]
                                 * rows[CROWS_ + r, pl.ds(v * 16, 16)])
                for sh in (8, 4, 2, 1):
                    acc = acc + _lane_shuffle(acc, lane ^ sh)
                out_vec = jnp.where(lane == r16, acc, out_vec)
            out_vec = 1.0 / (1.0 + jnp.exp(-out_vec))
            obuf[pl.ds(ch * CROWS_ + g * 16, 16)] = out_vec

    bufs = ((rows0, gs0), (rows1, gs1), (rows2, gs2), (rows3, gs3))
    for b, (rb, gb) in enumerate(bufs):
        _fire(b, rb, gb)

    def _quad(i, _):
        i4 = 4 * i
        for b, (rb, gb) in enumerate(bufs):
            _drain(rb, gb)
            _compute(i4 + b, rb)
            @pl.when(i4 + b + 4 < NCH_)
            def _():
                _fire(i4 + b + 4, rb, gb)
        return 0
    lax.fori_loop(0, NCH_ // 4, _quad, 0)

    pltpu.sync_copy(obuf, o_hbm.at[pl.ds(base, LPT_)])


def _classifier(cat, label_src, label_dst):
    kern = pl.kernel(
        _cls_body,
        out_type=jax.ShapeDtypeStruct((EL_P_,), jnp.float32),
        mesh=plsc.VectorSubcoreMesh(core_axis_name="c", subcore_axis_name="s"),
        scratch_types=[
            pltpu.VMEM((2 * LPT_,), jnp.int32),
            pltpu.VMEM((2 * LPT_,), jnp.int32),
            pltpu.VMEM((2 * CROWS_, H_), jnp.float32),
            pltpu.VMEM((2 * CROWS_, H_), jnp.float32),
            pltpu.VMEM((2 * CROWS_, H_), jnp.float32),
            pltpu.VMEM((2 * CROWS_, H_), jnp.float32),
            pltpu.VMEM((LPT_,), jnp.float32),
            pltpu.SemaphoreType.DMA,
            pltpu.SemaphoreType.DMA,
            pltpu.SemaphoreType.DMA,
            pltpu.SemaphoreType.DMA,
        ],
    )
    ls = jnp.pad(label_src, (0, EL_P_ - EL_))
    ld = jnp.pad(label_dst, (0, EL_P_ - EL_))
    return kern(cat, ls, ld)[:EL_]


def _enc_body(x_ref, w_ref, b_ref, e_ref, o_ref):
    o_ref[...] = (jnp.dot(x_ref[...], w_ref[...],
                          preferred_element_type=jnp.float32)
                  + b_ref[...] + e_ref[...])


def _encoder(prot_x, lin_W, lin_b, prot_emb):
    bm = 1000
    grid = NP_ // bm
    return pl.pallas_call(
        _enc_body,
        grid=(grid,),
        in_specs=[
            pl.BlockSpec((bm, DIN_), lambda i: (i, 0)),
            pl.BlockSpec((DIN_, H_), lambda i: (0, 0)),
            pl.BlockSpec((H_,), lambda i: (0,)),
            pl.BlockSpec((bm, H_), lambda i: (i, 0)),
        ],
        out_specs=pl.BlockSpec((bm, H_), lambda i: (i, 0)),
        out_shape=jax.ShapeDtypeStruct((NP_, H_), jnp.float32),
    )(prot_x, lin_W, lin_b, prot_emb)


def _gnn_body(a_ref, xp_ref, te_ref,
              wlt_ref, blt_ref, wrt_ref, wlp_ref, blp_ref, wrp_ref,
              xt2_out, xp2_out, acc_ref, cnt_ref, xps_ref, xtc_ref, xcur_ref):
    l = pl.program_id(0)
    j = pl.program_id(1)
    nj = pl.num_programs(1)
    a = a_ref[...]

    @pl.when((l == 0) & (j == 0))
    def _():
        cnt_ref[...] = jnp.zeros_like(cnt_ref)
        xtc_ref[...] = te_ref[...]

    @pl.when(j == 0)
    def _():
        acc_ref[...] = jnp.zeros_like(acc_ref)

    @pl.when(l == 0)
    def _():
        xcur_ref[...] = xp_ref[...]
        cnt_ref[...] += jnp.sum(a, axis=1)

    @pl.when(l == 1)
    def _():
        xcur_ref[...] = xps_ref[pl.ds(j * BJ_, BJ_), :]

    xpj = xcur_ref[...]
    acc_ref[...] += jnp.dot(a, xpj, preferred_element_type=jnp.float32)

    mp = lax.dot_general(a, xtc_ref[...],
                         dimension_numbers=(((0,), (0,)), ((), ())),
                         preferred_element_type=jnp.float32)
    cntp = jnp.sum(a, axis=0)
    aggp = mp * (1.0 / jnp.maximum(cntp, 1.0))[:, None]
    wlp = wlp_ref[...][0]
    wrp = wrp_ref[...][0]
    rp = (jnp.dot(aggp, wlp, preferred_element_type=jnp.float32)
          + jnp.where(l == 0, blp_ref[...][0], blp_ref[...][1])
          + jnp.dot(xpj, wrp, preferred_element_type=jnp.float32))

    @pl.when(l == 0)
    def _():
        xps_ref[pl.ds(j * BJ_, BJ_), :] = jnp.maximum(rp, 0.0)

    @pl.when(l == 1)
    def _():
        xp2_out[...] = rp

    @pl.when(j == nj - 1)
    def _():
        inv = 1.0 / jnp.maximum(cnt_ref[...], 1.0)
        aggt = acc_ref[...] * inv[:, None]
        rt = (jnp.dot(aggt, wlt_ref[...][0], preferred_element_type=jnp.float32)
              + jnp.where(l == 0, blt_ref[...][0], blt_ref[...][1])
              + jnp.dot(xtc_ref[...], wrt_ref[...][0],
                        preferred_element_type=jnp.float32))

        @pl.when(l == 0)
        def _():
            xtc_ref[...] = jnp.maximum(rt, 0.0)

        @pl.when(l == 1)
        def _():
            xt2_out[...] = rt


BJ_ = 1024


def _gnn(A, XP, TE, WLT, BLT, WRT, WLP, BLP, WRP):
    grid = (2, NPP_ // BJ_)
    return pl.pallas_call(
        _gnn_body,
        grid=grid,
        in_specs=[
            pl.BlockSpec((NT_, BJ_), lambda l, j: (0, j)),
            pl.BlockSpec((BJ_, H_), lambda l, j: (j * (1 - l), 0)),
            pl.BlockSpec((NT_, H_), lambda l, j: (0, 0)),
            pl.BlockSpec((1, H_, H_), lambda l, j: (l, 0, 0)),
            pl.BlockSpec((2, H_), lambda l, j: (0, 0)),
            pl.BlockSpec((1, H_, H_), lambda l, j: (l, 0, 0)),
            pl.BlockSpec((1, H_, H_), lambda l, j: (l, 0, 0)),
            pl.BlockSpec((2, H_), lambda l, j: (0, 0)),
            pl.BlockSpec((1, H_, H_), lambda l, j: (l, 0, 0)),
        ],
        out_specs=[
            pl.BlockSpec((NT_, H_), lambda l, j: (0, 0)),
            pl.BlockSpec((BJ_, H_), lambda l, j: (j, 0)),
        ],
        out_shape=[
            jax.ShapeDtypeStruct((NT_, H_), jnp.float32),
            jax.ShapeDtypeStruct((NPP_, H_), jnp.float32),
        ],
        scratch_shapes=[
            pltpu.VMEM((NT_, H_), jnp.float32),
            pltpu.VMEM((NT_,), jnp.float32),
            pltpu.VMEM((NPP_, H_), jnp.float32),
            pltpu.VMEM((NT_, H_), jnp.float32),
            pltpu.VMEM((BJ_, H_), jnp.float32),
        ],
    )(A, XP, TE, WLT, BLT, WRT, WLP, BLP, WRP)


def kernel(prot_x, prot_node_id, term_node_id, edge_src, edge_dst,
           label_src, label_dst, lin_W, lin_b, prot_emb, term_emb,
           c1_pt_Wl, c1_pt_bl, c1_pt_Wr, c1_tp_Wl, c1_tp_bl, c1_tp_Wr,
           c2_pt_Wl, c2_pt_bl, c2_pt_Wr, c2_tp_Wl, c2_tp_bl, c2_tp_Wr):
    edge_src = edge_src.astype(jnp.int32)
    edge_dst = edge_dst.astype(jnp.int32)
    label_src = label_src.astype(jnp.int32)
    label_dst = label_dst.astype(jnp.int32)

    a_flat = _build_adj(edge_dst, edge_src)
    A = a_flat.reshape(NT_, NPP_)

    xp = jnp.pad(_encoder(prot_x, lin_W, lin_b, prot_emb),
                 ((0, NPP_ - NP_), (0, 0)))
    WLT = jnp.stack([c1_pt_Wl, c2_pt_Wl])
    BLT = jnp.stack([c1_pt_bl, c2_pt_bl])
    WRT = jnp.stack([c1_pt_Wr, c2_pt_Wr])
    WLP = jnp.stack([c1_tp_Wl, c2_tp_Wl])
    BLP = jnp.stack([c1_tp_bl, c2_tp_bl])
    WRP = jnp.stack([c1_tp_Wr, c2_tp_Wr])
    xt2, xp2 = _gnn(A, xp, term_emb, WLT, BLT, WRT, WLP, BLP, WRP)

    cat = jnp.concatenate([xp2, xt2], axis=0)
    return _classifier(cat, label_src, label_dst)
